# baseline (speedup 1.0000x reference)
import jax
import jax.numpy as jnp
from jax.experimental import pallas as pl
from jax.experimental.pallas import tpu as pltpu

_BM = 200
_NBUF = 3


def _fused_body(adj_hbm, adjw_hbm, x_ref, w_ref, mlpw_ref, b_ref, o_ref,
                abuf, wbuf, asem, wsem):
    i = pl.program_id(0)
    nsteps = pl.num_programs(0)

    def _issue(step):
        slot = jax.lax.rem(step, _NBUF)
        pltpu.make_async_copy(
            adj_hbm.at[pl.ds(step * _BM, _BM), :], abuf.at[slot],
            asem.at[slot]).start()
        pltpu.make_async_copy(
            adjw_hbm.at[pl.ds(step * _BM, _BM), :], wbuf.at[slot],
            wsem.at[slot]).start()

    @pl.when(i == 0)
    def _prologue():
        _issue(0)
        _issue(1)

    @pl.when(i + _NBUF - 1 < nsteps)
    def _prefetch():
        _issue(i + _NBUF - 1)

    slot = jax.lax.rem(i, _NBUF)
    pltpu.make_async_copy(
        adj_hbm.at[pl.ds(i * _BM, _BM), :], abuf.at[slot], asem.at[slot]).wait()
    pltpu.make_async_copy(
        adjw_hbm.at[pl.ds(i * _BM, _BM), :], wbuf.at[slot], wsem.at[slot]).wait()

    xr = x_ref[:]
    h = (jnp.dot(abuf[slot], xr, preferred_element_type=jnp.float32)
         + jnp.dot(wbuf[slot], xr, preferred_element_type=jnp.float32))
    h = jnp.dot(h, w_ref[:], preferred_element_type=jnp.float32)
    norm = jnp.maximum(jnp.sqrt(jnp.sum(h * h, axis=-1, keepdims=True)), 1e-12)
    h = jnp.maximum(h / norm, 0.0)
    o_ref[:] = jax.lax.dot_general(
        h, mlpw_ref[:], (((1,), (1,)), ((), ())),
        preferred_element_type=jnp.float32) + b_ref[:]


def kernel(x, adj, adj_w, W, mlp_W, mlp_b):
    n, d = x.shape
    nclass = mlp_W.shape[0]
    b2 = mlp_b.reshape(1, nclass)

    return pl.pallas_call(
        _fused_body,
        grid=(n // _BM,),
        in_specs=[
            pl.BlockSpec(memory_space=pl.ANY),
            pl.BlockSpec(memory_space=pl.ANY),
            pl.BlockSpec((n, d), lambda i: (0, 0)),
            pl.BlockSpec((d, d), lambda i: (0, 0)),
            pl.BlockSpec((nclass, d), lambda i: (0, 0)),
            pl.BlockSpec((1, nclass), lambda i: (0, 0)),
        ],
        out_specs=pl.BlockSpec((_BM, nclass), lambda i: (i, 0)),
        out_shape=jax.ShapeDtypeStruct((n, nclass), jnp.float32),
        scratch_shapes=[
            pltpu.VMEM((_NBUF, _BM, n), jnp.float32),
            pltpu.VMEM((_NBUF, _BM, n), jnp.float32),
            pltpu.SemaphoreType.DMA((_NBUF,)),
            pltpu.SemaphoreType.DMA((_NBUF,)),
        ],
    )(adj, adj_w, x, W, mlp_W, b2)


# speedup vs baseline: 1.0005x; 1.0005x over previous
import jax
import jax.numpy as jnp
from jax.experimental import pallas as pl
from jax.experimental.pallas import tpu as pltpu

_BM = 200
_NBUF = 3


def _fused_body(adj_hbm, adjw_hbm, x_ref, w_ref, mlpw_ref, b_ref, o_ref,
                abuf, wbuf, asem, wsem):
    i = pl.program_id(0)
    nsteps = pl.num_programs(0)

    h1 = 96
    h2 = _BM - h1

    def _issue(step):
        slot = jax.lax.rem(step, _NBUF)
        pltpu.make_async_copy(
            adj_hbm.at[pl.ds(step * _BM, h1), :], abuf.at[slot, pl.ds(0, h1)],
            asem.at[slot, 0]).start()
        pltpu.make_async_copy(
            adjw_hbm.at[pl.ds(step * _BM, h1), :], wbuf.at[slot, pl.ds(0, h1)],
            wsem.at[slot, 0]).start()
        pltpu.make_async_copy(
            adj_hbm.at[pl.ds(step * _BM + h1, h2), :],
            abuf.at[slot, pl.ds(h1, h2)], asem.at[slot, 1]).start()
        pltpu.make_async_copy(
            adjw_hbm.at[pl.ds(step * _BM + h1, h2), :],
            wbuf.at[slot, pl.ds(h1, h2)], wsem.at[slot, 1]).start()

    @pl.when(i == 0)
    def _prologue():
        _issue(0)
        _issue(1)

    @pl.when(i + _NBUF - 1 < nsteps)
    def _prefetch():
        _issue(i + _NBUF - 1)

    slot = jax.lax.rem(i, _NBUF)
    pltpu.make_async_copy(
        adj_hbm.at[pl.ds(i * _BM, h1), :], abuf.at[slot, pl.ds(0, h1)],
        asem.at[slot, 0]).wait()
    pltpu.make_async_copy(
        adjw_hbm.at[pl.ds(i * _BM, h1), :], wbuf.at[slot, pl.ds(0, h1)],
        wsem.at[slot, 0]).wait()
    pltpu.make_async_copy(
        adj_hbm.at[pl.ds(i * _BM + h1, h2), :], abuf.at[slot, pl.ds(h1, h2)],
        asem.at[slot, 1]).wait()
    pltpu.make_async_copy(
        adjw_hbm.at[pl.ds(i * _BM + h1, h2), :], wbuf.at[slot, pl.ds(h1, h2)],
        wsem.at[slot, 1]).wait()

    xr = x_ref[:]
    h = (jnp.dot(abuf[slot], xr, preferred_element_type=jnp.float32)
         + jnp.dot(wbuf[slot], xr, preferred_element_type=jnp.float32))
    h = jnp.dot(h, w_ref[:], preferred_element_type=jnp.float32)
    norm = jnp.maximum(jnp.sqrt(jnp.sum(h * h, axis=-1, keepdims=True)), 1e-12)
    h = jnp.maximum(h / norm, 0.0)
    o_ref[:] = jax.lax.dot_general(
        h, mlpw_ref[:], (((1,), (1,)), ((), ())),
        preferred_element_type=jnp.float32) + b_ref[:]


def kernel(x, adj, adj_w, W, mlp_W, mlp_b):
    n, d = x.shape
    nclass = mlp_W.shape[0]
    b2 = mlp_b.reshape(1, nclass)

    return pl.pallas_call(
        _fused_body,
        grid=(n // _BM,),
        in_specs=[
            pl.BlockSpec(memory_space=pl.ANY),
            pl.BlockSpec(memory_space=pl.ANY),
            pl.BlockSpec((n, d), lambda i: (0, 0)),
            pl.BlockSpec((d, d), lambda i: (0, 0)),
            pl.BlockSpec((nclass, d), lambda i: (0, 0)),
            pl.BlockSpec((1, nclass), lambda i: (0, 0)),
        ],
        out_specs=pl.BlockSpec((_BM, nclass), lambda i: (i, 0)),
        out_shape=jax.ShapeDtypeStruct((n, nclass), jnp.float32),
        scratch_shapes=[
            pltpu.VMEM((_NBUF, _BM, n), jnp.float32),
            pltpu.VMEM((_NBUF, _BM, n), jnp.float32),
            pltpu.SemaphoreType.DMA((_NBUF, 2)),
            pltpu.SemaphoreType.DMA((_NBUF, 2)),
        ],
    )(adj, adj_w, x, W, mlp_W, b2)


# speedup vs baseline: 1.0128x; 1.0123x over previous
import jax
import jax.numpy as jnp
from jax.experimental import pallas as pl

_BM = 272


def _fused_body(adj_ref, adjw_ref, x_ref, w_ref, mlpw_ref, b_ref, o_ref):
    a = adj_ref[:] + adjw_ref[:]
    h = jnp.dot(a, x_ref[:], preferred_element_type=jnp.float32)
    h = jnp.dot(h, w_ref[:], preferred_element_type=jnp.float32)
    norm = jnp.maximum(jnp.sqrt(jnp.sum(h * h, axis=-1, keepdims=True)), 1e-12)
    h = jnp.maximum(h / norm, 0.0)
    o_ref[:] = jax.lax.dot_general(
        h, mlpw_ref[:], (((1,), (1,)), ((), ())),
        preferred_element_type=jnp.float32) + b_ref[:]


def kernel(x, adj, adj_w, W, mlp_W, mlp_b):
    n, d = x.shape
    nclass = mlp_W.shape[0]
    b2 = mlp_b.reshape(1, nclass)

    return pl.pallas_call(
        _fused_body,
        grid=(pl.cdiv(n, _BM),),
        in_specs=[
            pl.BlockSpec((_BM, n), lambda i: (i, 0)),
            pl.BlockSpec((_BM, n), lambda i: (i, 0)),
            pl.BlockSpec((n, d), lambda i: (0, 0)),
            pl.BlockSpec((d, d), lambda i: (0, 0)),
            pl.BlockSpec((nclass, d), lambda i: (0, 0)),
            pl.BlockSpec((1, nclass), lambda i: (0, 0)),
        ],
        out_specs=pl.BlockSpec((_BM, nclass), lambda i: (i, 0)),
        out_shape=jax.ShapeDtypeStruct((n, nclass), jnp.float32),
    )(adj, adj_w, x, W, mlp_W, b2)


# speedup vs baseline: 1.0183x; 1.0054x over previous
import jax
import jax.numpy as jnp
from jax.experimental import pallas as pl

_BM = 200


def _fused_body(adj_ref, adjw_ref, x_ref, w_ref, mlpw_ref, b_ref, o_ref):
    a = adj_ref[:] + adjw_ref[:]
    h = jnp.dot(a, x_ref[:], preferred_element_type=jnp.float32)
    h = jnp.dot(h, w_ref[:], preferred_element_type=jnp.float32,
                precision=jax.lax.Precision.HIGHEST)
    norm = jnp.maximum(jnp.sqrt(jnp.sum(h * h, axis=-1, keepdims=True)), 1e-12)
    h = jnp.maximum(h / norm, 0.0)
    o_ref[:] = jax.lax.dot_general(
        h, mlpw_ref[:], (((1,), (1,)), ((), ())),
        preferred_element_type=jnp.float32,
        precision=jax.lax.Precision.HIGHEST) + b_ref[:]


def kernel(x, adj, adj_w, W, mlp_W, mlp_b):
    n, d = x.shape
    nclass = mlp_W.shape[0]
    b2 = mlp_b.reshape(1, nclass)

    return pl.pallas_call(
        _fused_body,
        grid=(n // _BM,),
        in_specs=[
            pl.BlockSpec((_BM, n), lambda i: (i, 0)),
            pl.BlockSpec((_BM, n), lambda i: (i, 0)),
            pl.BlockSpec((n, d), lambda i: (0, 0)),
            pl.BlockSpec((d, d), lambda i: (0, 0)),
            pl.BlockSpec((nclass, d), lambda i: (0, 0)),
            pl.BlockSpec((1, nclass), lambda i: (0, 0)),
        ],
        out_specs=pl.BlockSpec((_BM, nclass), lambda i: (i, 0)),
        out_shape=jax.ShapeDtypeStruct((n, nclass), jnp.float32),
    )(adj, adj_w, x, W, mlp_W, b2)


# speedup vs baseline: 1.0211x; 1.0028x over previous
import jax
import jax.numpy as jnp
from jax.experimental import pallas as pl

_BM = 200


def _fused_body(adj_ref, adjw_ref, x_ref, w_ref, mlpw_ref, b_ref, o_ref):
    a = adj_ref[:] + adjw_ref[:]
    h = jnp.dot(a, x_ref[:], preferred_element_type=jnp.float32)
    h = jnp.dot(h, w_ref[:], preferred_element_type=jnp.float32)
    norm = jnp.maximum(jnp.sqrt(jnp.sum(h * h, axis=-1, keepdims=True)), 1e-12)
    h = jnp.maximum(h / norm, 0.0)
    o_ref[:] = jax.lax.dot_general(
        h, mlpw_ref[:], (((1,), (1,)), ((), ())),
        preferred_element_type=jnp.float32) + b_ref[:]


def kernel(x, adj, adj_w, W, mlp_W, mlp_b):
    n, d = x.shape
    nclass = mlp_W.shape[0]
    b2 = mlp_b.reshape(1, nclass)

    return pl.pallas_call(
        _fused_body,
        grid=(n // _BM,),
        in_specs=[
            pl.BlockSpec((_BM, n), lambda i: (i, 0)),
            pl.BlockSpec((_BM, n), lambda i: (i, 0)),
            pl.BlockSpec((n, d), lambda i: (0, 0)),
            pl.BlockSpec((d, d), lambda i: (0, 0)),
            pl.BlockSpec((nclass, d), lambda i: (0, 0)),
            pl.BlockSpec((1, nclass), lambda i: (0, 0)),
        ],
        out_specs=pl.BlockSpec((_BM, nclass), lambda i: (i, 0)),
        out_shape=jax.ShapeDtypeStruct((n, nclass), jnp.float32),
    )(adj, adj_w, x, W, mlp_W, b2)
